# baseline (speedup 1.0000x reference)
import functools

import jax
import jax.numpy as jnp
from jax.experimental import pallas as pl
from jax.experimental.pallas import tpu as pltpu


def _round_up(a, b):
    return (a + b - 1) // b * b


def _ln(y, w, b, eps=1e-5):
    mu = jnp.mean(y, axis=-1, keepdims=True)
    yc = y - mu
    var = jnp.mean(yc * yc, axis=-1, keepdims=True)
    return yc * jax.lax.rsqrt(var + eps) * w + b


def _fused_kernel(count_ref, mean_ref, sqsum_ref, x_ref,
                  we_ref, be_ref, ln1w_ref, ln1b_ref,
                  w1_ref, b1_ref, w2_ref, b2_ref, ln2w_ref, ln2b_ref,
                  o_ref, mask_ref, *, cliprange):
    bb, items, d_in = x_ref.shape
    x3 = x_ref[...]
    x = x3.reshape(bb * items, d_in)
    empty = x[:, 0:1] == 0.0
    keep = jnp.where(empty, 0.0, 1.0)

    count = count_ref[0]
    denom = jnp.maximum(count - 1.0, 1.0)
    var = sqsum_ref[...] / denom
    inv_sd = jnp.where(var != 0.0, jax.lax.rsqrt(var), 1.0)
    use_norm = count > 1.0
    scale = jnp.where(use_norm, inv_sd, 1.0)
    shift = jnp.where(use_norm, mean_ref[...], 0.0)
    xn = jnp.clip((x - shift) * scale, -cliprange, cliprange)

    h = jnp.dot(xn.astype(jnp.bfloat16), we_ref[...],
                preferred_element_type=jnp.float32) + be_ref[...]
    h = _ln(jnp.maximum(h, 0.0), ln1w_ref[...], ln1b_ref[...])
    f = jnp.dot(h.astype(jnp.bfloat16), w1_ref[...],
                preferred_element_type=jnp.float32) + b1_ref[...]
    f = jnp.maximum(f, 0.0)
    r = jnp.dot(f.astype(jnp.bfloat16), w2_ref[...],
                preferred_element_type=jnp.float32) + b2_ref[...]
    r = jnp.maximum(r, 0.0)
    h = _ln(h + r, ln2w_ref[...], ln2b_ref[...])
    o_ref[...] = (h * keep).astype(o_ref.dtype).reshape(o_ref.shape)
    mask_ref[...] = empty.reshape(bb, items)


def kernel(x, mean, squares_sum, count, w_emb, b_emb, ln1_w, ln1_b,
           w_ff1, b_ff1, w_ff2, b_ff2, ln2_w, ln2_b, *, block_batch=8):
    B, items, d_in = x.shape
    d_model = w_emb.shape[1]

    bb = min(block_batch, B)
    grid = pl.cdiv(B, bb)
    B_pad = grid * bb
    if B_pad != B:
        x = jnp.pad(x, ((0, B_pad - B), (0, 0), (0, 0)))

    count_arr = jnp.asarray([count], dtype=jnp.float32)
    mean_r = mean.astype(jnp.float32).reshape(1, d_in)
    sqsum_r = squares_sum.astype(jnp.float32).reshape(1, d_in)

    weights = [w_emb.astype(jnp.bfloat16), b_emb, ln1_w, ln1_b,
               w_ff1.astype(jnp.bfloat16), b_ff1,
               w_ff2.astype(jnp.bfloat16), b_ff2, ln2_w, ln2_b]
    weight_specs = [pl.BlockSpec(tuple(w.shape), lambda i, cnt: (0, 0))
                    for w in weights]

    y, mask = pl.pallas_call(
        functools.partial(_fused_kernel, cliprange=5.0),
        out_shape=(jax.ShapeDtypeStruct((B_pad, items, d_model), jnp.float32),
                   jax.ShapeDtypeStruct((B_pad, items), jnp.bool_)),
        grid_spec=pltpu.PrefetchScalarGridSpec(
            num_scalar_prefetch=1,
            grid=(grid,),
            in_specs=[
                pl.BlockSpec((1, d_in), lambda i, cnt: (0, 0)),
                pl.BlockSpec((1, d_in), lambda i, cnt: (0, 0)),
                pl.BlockSpec((bb, items, d_in), lambda i, cnt: (i, 0, 0)),
            ] + weight_specs,
            out_specs=(
                pl.BlockSpec((bb, items, d_model), lambda i, cnt: (i, 0, 0)),
                pl.BlockSpec((bb, items), lambda i, cnt: (i, 0)),
            ),
        ),
        compiler_params=pltpu.CompilerParams(
            dimension_semantics=("parallel",),
            vmem_limit_bytes=64 * 1024 * 1024,
        ),
    )(count_arr, mean_r, sqsum_r, x, *weights)

    if B_pad != B:
        y = y[:B]
        mask = mask[:B]
    return y, None, mask


# speedup vs baseline: 1.1639x; 1.1639x over previous
import functools

import jax
import jax.numpy as jnp
from jax.experimental import pallas as pl
from jax.experimental.pallas import tpu as pltpu


def _round_up(a, b):
    return (a + b - 1) // b * b


def _ln(y, w, b, eps=1e-5):
    mu = jnp.mean(y, axis=-1, keepdims=True)
    yc = y - mu
    var = jnp.mean(yc * yc, axis=-1, keepdims=True)
    return yc * jax.lax.rsqrt(var + eps) * w + b


def _fused_kernel(count_ref, mean_ref, sqsum_ref, x_ref,
                  we_ref, be_ref, ln1w_ref, ln1b_ref,
                  w1_ref, b1_ref, w2_ref, b2_ref, ln2w_ref, ln2b_ref,
                  o_ref, mask_ref, *, cliprange):
    bb, items, d_in = x_ref.shape
    x3 = x_ref[...]
    x = x3.reshape(bb * items, d_in)
    empty = x[:, 0:1] == 0.0
    keep = jnp.where(empty, 0.0, 1.0)

    count = count_ref[0]
    denom = jnp.maximum(count - 1.0, 1.0)
    var = sqsum_ref[...] / denom
    inv_sd = jnp.where(var != 0.0, jax.lax.rsqrt(var), 1.0)
    use_norm = count > 1.0
    scale = jnp.where(use_norm, inv_sd, 1.0)
    shift = jnp.where(use_norm, mean_ref[...], 0.0)
    xn = jnp.clip((x - shift) * scale, -cliprange, cliprange)

    h = jnp.dot(xn.astype(jnp.bfloat16), we_ref[...],
                preferred_element_type=jnp.float32) + be_ref[...]
    h = _ln(jnp.maximum(h, 0.0), ln1w_ref[...], ln1b_ref[...])
    f = jnp.dot(h.astype(jnp.bfloat16), w1_ref[...],
                preferred_element_type=jnp.float32) + b1_ref[...]
    f = jnp.maximum(f, 0.0)
    r = jnp.dot(f.astype(jnp.bfloat16), w2_ref[...],
                preferred_element_type=jnp.float32) + b2_ref[...]
    r = jnp.maximum(r, 0.0)
    h = _ln(h + r, ln2w_ref[...], ln2b_ref[...])
    o_ref[...] = (h * keep).astype(o_ref.dtype).reshape(o_ref.shape)
    mask_ref[...] = empty.reshape(bb, items)


def kernel(x, mean, squares_sum, count, w_emb, b_emb, ln1_w, ln1_b,
           w_ff1, b_ff1, w_ff2, b_ff2, ln2_w, ln2_b, *, block_batch=32):
    B, items, d_in = x.shape
    d_model = w_emb.shape[1]

    bb = min(block_batch, B)
    grid = pl.cdiv(B, bb)
    B_pad = grid * bb
    if B_pad != B:
        x = jnp.pad(x, ((0, B_pad - B), (0, 0), (0, 0)))

    count_arr = jnp.asarray([count], dtype=jnp.float32)
    mean_r = mean.astype(jnp.float32).reshape(1, d_in)
    sqsum_r = squares_sum.astype(jnp.float32).reshape(1, d_in)

    weights = [w_emb.astype(jnp.bfloat16), b_emb, ln1_w, ln1_b,
               w_ff1.astype(jnp.bfloat16), b_ff1,
               w_ff2.astype(jnp.bfloat16), b_ff2, ln2_w, ln2_b]
    weight_specs = [pl.BlockSpec(tuple(w.shape), lambda i, cnt: (0, 0))
                    for w in weights]

    y, mask = pl.pallas_call(
        functools.partial(_fused_kernel, cliprange=5.0),
        out_shape=(jax.ShapeDtypeStruct((B_pad, items, d_model), jnp.float32),
                   jax.ShapeDtypeStruct((B_pad, items), jnp.bool_)),
        grid_spec=pltpu.PrefetchScalarGridSpec(
            num_scalar_prefetch=1,
            grid=(grid,),
            in_specs=[
                pl.BlockSpec((1, d_in), lambda i, cnt: (0, 0)),
                pl.BlockSpec((1, d_in), lambda i, cnt: (0, 0)),
                pl.BlockSpec((bb, items, d_in), lambda i, cnt: (i, 0, 0)),
            ] + weight_specs,
            out_specs=(
                pl.BlockSpec((bb, items, d_model), lambda i, cnt: (i, 0, 0)),
                pl.BlockSpec((bb, items), lambda i, cnt: (i, 0)),
            ),
        ),
        compiler_params=pltpu.CompilerParams(
            dimension_semantics=("parallel",),
            vmem_limit_bytes=64 * 1024 * 1024,
        ),
    )(count_arr, mean_r, sqsum_r, x, *weights)

    if B_pad != B:
        y = y[:B]
        mask = mask[:B]
    return y, None, mask


# speedup vs baseline: 1.1847x; 1.0178x over previous
import functools

import jax
import jax.numpy as jnp
from jax.experimental import pallas as pl
from jax.experimental.pallas import tpu as pltpu


def _round_up(a, b):
    return (a + b - 1) // b * b


def _ln(y, w, b, eps=1e-5):
    mu = jnp.mean(y, axis=-1, keepdims=True)
    yc = y - mu
    var = jnp.mean(yc * yc, axis=-1, keepdims=True)
    return yc * jax.lax.rsqrt(var + eps) * w + b


def _fused_kernel(count_ref, mean_ref, sqsum_ref, x_ref,
                  we_ref, be_ref, ln1w_ref, ln1b_ref,
                  w1_ref, b1_ref, w2_ref, b2_ref, ln2w_ref, ln2b_ref,
                  o_ref, mask_ref, *, cliprange):
    bb, items, d_in = x_ref.shape
    x3 = x_ref[...]
    x = x3.reshape(bb * items, d_in)
    empty = x[:, 0:1] == 0.0
    keep = jnp.where(empty, 0.0, 1.0)

    count = count_ref[0]
    denom = jnp.maximum(count - 1.0, 1.0)
    var = sqsum_ref[...] / denom
    inv_sd = jnp.where(var != 0.0, jax.lax.rsqrt(var), 1.0)
    use_norm = count > 1.0
    scale = jnp.where(use_norm, inv_sd, 1.0)
    shift = jnp.where(use_norm, mean_ref[...], 0.0)
    xn = jnp.clip((x - shift) * scale, -cliprange, cliprange)

    h = jnp.dot(xn.astype(jnp.bfloat16), we_ref[...],
                preferred_element_type=jnp.float32) + be_ref[...]
    h = _ln(jnp.maximum(h, 0.0), ln1w_ref[...], ln1b_ref[...])
    f = jnp.dot(h.astype(jnp.bfloat16), w1_ref[...],
                preferred_element_type=jnp.float32) + b1_ref[...]
    f = jnp.maximum(f, 0.0)
    r = jnp.dot(f.astype(jnp.bfloat16), w2_ref[...],
                preferred_element_type=jnp.float32) + b2_ref[...]
    r = jnp.maximum(r, 0.0)
    h = _ln(h + r, ln2w_ref[...], ln2b_ref[...])
    o_ref[...] = (h * keep).astype(o_ref.dtype).reshape(o_ref.shape)
    mask_ref[...] = empty.reshape(bb, items)


def kernel(x, mean, squares_sum, count, w_emb, b_emb, ln1_w, ln1_b,
           w_ff1, b_ff1, w_ff2, b_ff2, ln2_w, ln2_b, *, block_batch=128):
    B, items, d_in = x.shape
    d_model = w_emb.shape[1]

    bb = min(block_batch, B)
    grid = pl.cdiv(B, bb)
    B_pad = grid * bb
    if B_pad != B:
        x = jnp.pad(x, ((0, B_pad - B), (0, 0), (0, 0)))

    count_arr = jnp.asarray([count], dtype=jnp.float32)
    mean_r = mean.astype(jnp.float32).reshape(1, d_in)
    sqsum_r = squares_sum.astype(jnp.float32).reshape(1, d_in)

    weights = [w_emb.astype(jnp.bfloat16), b_emb, ln1_w, ln1_b,
               w_ff1.astype(jnp.bfloat16), b_ff1,
               w_ff2.astype(jnp.bfloat16), b_ff2, ln2_w, ln2_b]
    weight_specs = [pl.BlockSpec(tuple(w.shape), lambda i, cnt: (0, 0))
                    for w in weights]

    y, mask = pl.pallas_call(
        functools.partial(_fused_kernel, cliprange=5.0),
        out_shape=(jax.ShapeDtypeStruct((B_pad, items, d_model), jnp.float32),
                   jax.ShapeDtypeStruct((B_pad, items), jnp.bool_)),
        grid_spec=pltpu.PrefetchScalarGridSpec(
            num_scalar_prefetch=1,
            grid=(grid,),
            in_specs=[
                pl.BlockSpec((1, d_in), lambda i, cnt: (0, 0)),
                pl.BlockSpec((1, d_in), lambda i, cnt: (0, 0)),
                pl.BlockSpec((bb, items, d_in), lambda i, cnt: (i, 0, 0)),
            ] + weight_specs,
            out_specs=(
                pl.BlockSpec((bb, items, d_model), lambda i, cnt: (i, 0, 0)),
                pl.BlockSpec((bb, items), lambda i, cnt: (i, 0)),
            ),
        ),
        compiler_params=pltpu.CompilerParams(
            dimension_semantics=("parallel",),
            vmem_limit_bytes=64 * 1024 * 1024,
        ),
    )(count_arr, mean_r, sqsum_r, x, *weights)

    if B_pad != B:
        y = y[:B]
        mask = mask[:B]
    return y, None, mask


# speedup vs baseline: 1.6010x; 1.3515x over previous
import functools

import jax
import jax.numpy as jnp
from jax.experimental import pallas as pl
from jax.experimental.pallas import tpu as pltpu


def _round_up(a, b):
    return (a + b - 1) // b * b


def _ln(y, w, b, eps=1e-5):
    mu = jnp.mean(y, axis=-1, keepdims=True)
    yc = y - mu
    var = jnp.mean(yc * yc, axis=-1, keepdims=True)
    return yc * jax.lax.rsqrt(var + eps) * w + b


def _fused_kernel(count_ref, mean_ref, sqsum_ref, x_ref,
                  we_ref, be_ref, ln1w_ref, ln1b_ref,
                  w1_ref, b1_ref, w2_ref, b2_ref, ln2w_ref, ln2b_ref,
                  o_ref, *, cliprange):
    x = x_ref[...]
    keep = jnp.where(x[:, 0:1] == 0.0, 0.0, 1.0)

    count = count_ref[0]
    denom = jnp.maximum(count - 1.0, 1.0)
    var = sqsum_ref[...] / denom
    inv_sd = jnp.where(var != 0.0, jax.lax.rsqrt(var), 1.0)
    use_norm = count > 1.0
    scale = jnp.where(use_norm, inv_sd, 1.0)
    shift = jnp.where(use_norm, mean_ref[...], 0.0)
    xn = jnp.clip((x - shift) * scale, -cliprange, cliprange)

    h = jnp.dot(xn.astype(jnp.bfloat16), we_ref[...],
                preferred_element_type=jnp.float32) + be_ref[...]
    h = _ln(jnp.maximum(h, 0.0), ln1w_ref[...], ln1b_ref[...])
    f = jnp.dot(h.astype(jnp.bfloat16), w1_ref[...],
                preferred_element_type=jnp.float32) + b1_ref[...]
    f = jnp.maximum(f, 0.0)
    r = jnp.dot(f.astype(jnp.bfloat16), w2_ref[...],
                preferred_element_type=jnp.float32) + b2_ref[...]
    r = jnp.maximum(r, 0.0)
    h = _ln(h + r, ln2w_ref[...], ln2b_ref[...])
    o_ref[...] = (h * keep).astype(o_ref.dtype)


def kernel(x, mean, squares_sum, count, w_emb, b_emb, ln1_w, ln1_b,
           w_ff1, b_ff1, w_ff2, b_ff2, ln2_w, ln2_b, *, block_rows=4096):
    B, items, d_in = x.shape
    d_model = w_emb.shape[1]
    R = B * items
    x2 = x.reshape(R, d_in)

    tr = _round_up(min(block_rows, _round_up(R, 8)), 8)
    R_pad = _round_up(R, tr)
    if R_pad != R:
        x2 = jnp.pad(x2, ((0, R_pad - R), (0, 0)))

    count_arr = jnp.asarray([count], dtype=jnp.float32)
    mean_r = mean.astype(jnp.float32).reshape(1, d_in)
    sqsum_r = squares_sum.astype(jnp.float32).reshape(1, d_in)

    weights = [w_emb.astype(jnp.bfloat16), b_emb, ln1_w, ln1_b,
               w_ff1.astype(jnp.bfloat16), b_ff1,
               w_ff2.astype(jnp.bfloat16), b_ff2, ln2_w, ln2_b]
    weight_specs = [pl.BlockSpec(tuple(w.shape), lambda i, cnt: (0, 0))
                    for w in weights]

    out = pl.pallas_call(
        functools.partial(_fused_kernel, cliprange=5.0),
        out_shape=jax.ShapeDtypeStruct((R_pad, d_model), jnp.float32),
        grid_spec=pltpu.PrefetchScalarGridSpec(
            num_scalar_prefetch=1,
            grid=(R_pad // tr,),
            in_specs=[
                pl.BlockSpec((1, d_in), lambda i, cnt: (0, 0)),
                pl.BlockSpec((1, d_in), lambda i, cnt: (0, 0)),
                pl.BlockSpec((tr, d_in), lambda i, cnt: (i, 0)),
            ] + weight_specs,
            out_specs=pl.BlockSpec((tr, d_model), lambda i, cnt: (i, 0)),
        ),
        compiler_params=pltpu.CompilerParams(
            dimension_semantics=("parallel",),
            vmem_limit_bytes=64 * 1024 * 1024,
        ),
    )(count_arr, mean_r, sqsum_r, x2, *weights)

    y = out[:R].reshape(B, items, d_model)
    mask = x[:, :, 0] == 0
    return y, None, mask
